# baseline (speedup 1.0000x reference)
import functools

import jax
import jax.numpy as jnp
from jax.experimental import pallas as pl
from jax.experimental.pallas import tpu as pltpu
from jax.experimental.pallas import tpu_sc as plsc

_LANES = 16


@jax.jit
def kernel(x, token_table, pos_table):
    B, L = x.shape
    V, D = token_table.shape
    N = B * L
    W = 8 * L

    x_flat = x.reshape(N).astype(jnp.int32)

    @functools.partial(
        pl.kernel,
        out_type=jax.ShapeDtypeStruct((N, D), jnp.float32),
        mesh=plsc.VectorSubcoreMesh(
            core_axis_name="core", subcore_axis_name="subcore"
        ),
        scratch_types=[pltpu.VMEM((L, D), jnp.float32)],
        compiler_params=pltpu.CompilerParams(use_tc_tiling_on_sc=False),
    )
    def sc_embed(tok_hbm, idx_hbm, pos_hbm, out_hbm, pos_vmem):
        pltpu.sync_copy(pos_hbm, pos_vmem)

        def body(i_vmem, o_vmem):
            pltpu.sync_copy(tok_hbm.at[i_vmem], o_vmem)

            @pl.loop(0, L)
            def _(l):
                for c in range(0, D, _LANES):
                    p = pos_vmem[l, pl.ds(c, _LANES)]
                    for s in range(W // L):
                        plsc.addupdate(
                            o_vmem.at[s * L + l, pl.ds(c, _LANES)], p
                        )

        pltpu.emit_pipeline(
            body,
            grid=(N // W,),
            in_specs=[pl.BlockSpec((W,), lambda i: (i,))],
            out_specs=[pl.BlockSpec((W, D), lambda i: (i, 0))],
            core_axis_name=("core", "subcore"),
            dimension_semantics=(pltpu.PARALLEL,),
        )(idx_hbm, out_hbm)

    flat = sc_embed(token_table, x_flat, pos_table)

    x3 = flat.reshape(B, L, D)
    BB, LB = 512, 8

    def tc_body(in_ref, out_ref):
        out_ref[...] = jnp.transpose(in_ref[...], (1, 2, 0))

    out3 = pl.pallas_call(
        tc_body,
        grid=(B // BB, L // LB),
        in_specs=[pl.BlockSpec((BB, LB, D), lambda i, j: (i, j, 0))],
        out_specs=pl.BlockSpec((LB, D, BB), lambda i, j: (j, 0, i)),
        out_shape=jax.ShapeDtypeStruct((L, D, B), jnp.float32),
    )(x3)
    return out3.transpose(2, 0, 1)


# speedup vs baseline: 2.5233x; 2.5233x over previous
import functools

import jax
import jax.numpy as jnp
from jax.experimental import pallas as pl
from jax.experimental.pallas import tpu as pltpu
from jax.experimental.pallas import tpu_sc as plsc

_LANES = 16


@jax.jit
def kernel(x, token_table, pos_table):
    B, L = x.shape
    V, D = token_table.shape
    N = B * L
    W = 8 * L

    x_flat = x.reshape(N).astype(jnp.int32)

    @functools.partial(
        pl.kernel,
        out_type=jax.ShapeDtypeStruct((N, D), jnp.float32),
        mesh=plsc.VectorSubcoreMesh(
            core_axis_name="core", subcore_axis_name="subcore"
        ),
        scratch_types=[pltpu.VMEM((L, D), jnp.float32)],
        compiler_params=pltpu.CompilerParams(use_tc_tiling_on_sc=False),
    )
    def sc_embed(tok_hbm, idx_hbm, pos_hbm, out_hbm, pos_vmem):
        pltpu.sync_copy(pos_hbm, pos_vmem)

        def body(i_vmem, o_vmem):
            pltpu.sync_copy(tok_hbm.at[i_vmem], o_vmem)

            @pl.loop(0, L)
            def _(l):
                for c in range(0, D, _LANES):
                    p = pos_vmem[l, pl.ds(c, _LANES)]
                    for s in range(W // L):
                        plsc.addupdate(
                            o_vmem.at[s * L + l, pl.ds(c, _LANES)], p
                        )

        pltpu.emit_pipeline(
            body,
            grid=(N // W,),
            in_specs=[pl.BlockSpec((W,), lambda i: (i,))],
            out_specs=[pl.BlockSpec((W, D), lambda i: (i, 0))],
            core_axis_name=("core", "subcore"),
            dimension_semantics=(pltpu.PARALLEL,),
        )(idx_hbm, out_hbm)

    flat = sc_embed(token_table, x_flat, pos_table)

    G = L // 4
    t2 = flat.reshape(N // 4, 128)
    BB = 256

    def tc_body(in_ref, out_ref):
        v = in_ref[...].reshape(BB, G, 128)
        for g in range(G):
            out_ref[g] = v[:, g, :].T

    out128 = pl.pallas_call(
        tc_body,
        grid=(B // BB,),
        in_specs=[pl.BlockSpec((G * BB, 128), lambda i: (i, 0))],
        out_specs=pl.BlockSpec((G, 128, BB), lambda i: (0, 0, i)),
        out_shape=jax.ShapeDtypeStruct((G, 128, B), jnp.float32),
    )(t2)
    return out128.reshape(L, D, B).transpose(2, 0, 1)


# speedup vs baseline: 2.6329x; 1.0434x over previous
import functools

import jax
import jax.numpy as jnp
from jax.experimental import pallas as pl
from jax.experimental.pallas import tpu as pltpu
from jax.experimental.pallas import tpu_sc as plsc

_LANES = 16


@jax.jit
def kernel(x, token_table, pos_table):
    B, L = x.shape
    V, D = token_table.shape
    N = B * L
    W = 8 * L

    x_flat = x.reshape(N).astype(jnp.int32)

    @functools.partial(
        pl.kernel,
        out_type=jax.ShapeDtypeStruct((N, D), jnp.float32),
        mesh=plsc.VectorSubcoreMesh(
            core_axis_name="core", subcore_axis_name="subcore"
        ),
        compiler_params=pltpu.CompilerParams(use_tc_tiling_on_sc=False),
    )
    def sc_embed(tok_hbm, idx_hbm, out_hbm):
        def body(i_vmem, o_vmem):
            pltpu.sync_copy(tok_hbm.at[i_vmem], o_vmem)

        pltpu.emit_pipeline(
            body,
            grid=(N // W,),
            in_specs=[pl.BlockSpec((W,), lambda i: (i,))],
            out_specs=[pl.BlockSpec((W, D), lambda i: (i, 0))],
            core_axis_name=("core", "subcore"),
            dimension_semantics=(pltpu.PARALLEL,),
        )(idx_hbm, out_hbm)

    flat = sc_embed(token_table, x_flat)

    G = L // 4
    t2 = flat.reshape(N // 4, 128)
    BB = 256

    pos128 = pos_table.reshape(G, 128)

    def tc_body(in_ref, pos_ref, out_ref):
        v = in_ref[...].reshape(BB, G, 128)
        for g in range(G):
            out_ref[g] = v[:, g, :].T + pos_ref[g][:, None]

    out128 = pl.pallas_call(
        tc_body,
        grid=(B // BB,),
        in_specs=[
            pl.BlockSpec((G * BB, 128), lambda i: (i, 0)),
            pl.BlockSpec((G, 128), lambda i: (0, 0)),
        ],
        out_specs=pl.BlockSpec((G, 128, BB), lambda i: (0, 0, i)),
        out_shape=jax.ShapeDtypeStruct((G, 128, B), jnp.float32),
    )(t2, pos128)
    return out128.reshape(L, D, B).transpose(2, 0, 1)
